# baseline (speedup 1.0000x reference)
import functools

import jax
import jax.numpy as jnp
from jax.experimental import pallas as pl
from jax.experimental.pallas import tpu as pltpu

T = 2048
C = 128
K = 3
CYC = 64
EPS_GN = 1e-5


def _erf(x):
    s = jnp.sign(x)
    a = jnp.abs(x)
    t = 1.0 / (1.0 + 0.3275911 * a)
    poly = ((((1.061405429 * t - 1.453152027) * t + 1.421413741) * t
             - 0.284496736) * t + 0.254829592) * t
    return s * (1.0 - poly * jnp.exp(-a * a))


def _gelu_exact(x):
    return 0.5 * x * (1.0 + _erf(x * 0.7071067811865476))


def _body(x_ref, wdw_ref, wpwt_ref, gng_ref, gnb_ref, wgt_ref, res_ref,
          wloc_ref, wlocpwt_ref, out_ref):
    A = x_ref[0]
    W_dw9 = wdw_ref[...]
    W_pwT = wpwt_ref[...]
    gn_gamma = gng_ref[...]
    gn_beta = gnb_ref[...]
    W_gateT = wgt_ref[...]
    res_row = res_ref[...]
    W_loc7 = wloc_ref[...]
    W_loc_pwT = wlocpwt_ref[...]

    f32 = jnp.float32

    Apad = jnp.concatenate([A, A[:128]], axis=0)
    S = jnp.zeros((128, 256), f32)
    for i in range(16):
        Ai = A[128 * i:128 * (i + 1)]
        Bi = Apad[128 * i:128 * i + 256]
        S = S + jax.lax.dot_general(Ai, Bi, (((1,), (1,)), ((), ())),
                                    preferred_element_type=f32)
    row = jax.lax.broadcasted_iota(jnp.int32, (128, 256), 0)
    for k in range(7):
        s = 1 << k
        rolled = jnp.concatenate([S[:, s:], S[:, :s]], axis=1)
        S = jnp.where((row & s) != 0, rolled, S)
    r_full = jnp.sum(S, axis=0, keepdims=True) * (1.0 / C)

    lag = jax.lax.broadcasted_iota(jnp.int32, (1, 256), 1)
    NEG = f32(-3e38)
    rm = jnp.where((lag >= 32) & (lag <= 128), r_full, NEG)
    vs, ids = [], []
    for _ in range(K):
        v = jnp.max(rm, axis=1, keepdims=True)
        i_k = jnp.min(jnp.where(rm >= v, lag, 9999), axis=1, keepdims=True)
        rm = jnp.where(lag == i_k, NEG, rm)
        vs.append(v)
        ids.append(i_k)
    vmax = jnp.maximum(jnp.maximum(vs[0], vs[1]), vs[2])
    es = [jnp.exp(v - vmax) for v in vs]
    ssum = es[0] + es[1] + es[2]
    ws = [e / ssum for e in es]
    H = -(ws[0] * jnp.log(ws[0] + 1e-8) + ws[1] * jnp.log(ws[1] + 1e-8)
          + ws[2] * jnp.log(ws[2] + 1e-8))
    Hmax = jnp.log(f32(K) + 1e-8)
    Gamma = jnp.clip(1.0 - H / (Hmax + 1e-8), 0.0, 1.0)
    gate = jax.nn.sigmoid(4.0 * (Gamma - 0.5))

    acc = jnp.zeros((T, C), f32)
    for k in range(K):
        p_i = ids[k]
        p_f = p_i.astype(f32)
        tpad = (jnp.ceil(2048.0 / p_f) * p_f).astype(jnp.int32)
        cyc_i = jax.lax.broadcasted_iota(jnp.int32, (CYC, T), 0)
        t_i = jax.lax.broadcasted_iota(jnp.int32, (CYC, T), 1)
        q = cyc_i * p_i
        in_win = (t_i >= q) & (t_i < q + p_i)
        tt = 2 * (T - 1) - t_i
        refl = (tt >= T) & (tt < tpad) & (tt >= q) & (tt < q + p_i)
        M = in_win.astype(f32) + refl.astype(f32)
        u = jnp.dot(M, A, preferred_element_type=f32) * (1.0 / 128.0)

        upad = jnp.concatenate(
            [jnp.zeros((4, C), f32), u, jnp.zeros((4, C), f32)], axis=0)
        xc = jnp.zeros((CYC, C), f32)
        for j in range(9):
            xc = xc + upad[j:j + CYC] * W_dw9[j][None, :]
        xp = jnp.dot(xc, W_pwT, preferred_element_type=f32)

        g_of_c = jax.lax.broadcasted_iota(jnp.int32, (C, 32), 0) // 4
        g_id = jax.lax.broadcasted_iota(jnp.int32, (C, 32), 1)
        G = (g_of_c == g_id).astype(f32)
        colsum = jnp.sum(xp, axis=0, keepdims=True)
        mu_g = jnp.dot(colsum, G) * (1.0 / 256.0)
        mu_row = jnp.dot(mu_g, G.T)
        dev = xp - mu_row
        var_g = jnp.dot(jnp.sum(dev * dev, axis=0, keepdims=True), G) * (1.0 / 256.0)
        var_row = jnp.dot(var_g, G.T)
        xn = dev * jax.lax.rsqrt(var_row + EPS_GN)
        xn = xn * gn_gamma + gn_beta
        xg = _gelu_exact(xn)

        ubar = jnp.sum(u, axis=0, keepdims=True) * (1.0 / 64.0)
        gv = jax.nn.sigmoid(jnp.dot(ubar, W_gateT))
        xconv = xg * gv

        cyc2 = jax.lax.broadcasted_iota(jnp.int32, (T, CYC), 1)
        t2 = jax.lax.broadcasted_iota(jnp.int32, (T, CYC), 0)
        q2 = cyc2 * p_i
        MT = ((t2 >= q2) & (t2 < q2 + p_i)).astype(f32)
        acc = acc + ws[k] * jnp.dot(MT, xconv, preferred_element_type=f32)

    periodic = A + res_row * acc

    xpad = jnp.concatenate(
        [jnp.zeros((3, C), f32), A, jnp.zeros((3, C), f32)], axis=0)
    loc = jnp.zeros((T, C), f32)
    for j in range(7):
        loc = loc + xpad[j:j + T] * W_loc7[j][None, :]
    lp = jnp.dot(loc, W_loc_pwT, preferred_element_type=f32)
    local = _gelu_exact(lp)

    num = jnp.sum(local * periodic, axis=0, keepdims=True)
    den = jnp.sum(periodic * periodic, axis=0, keepdims=True) + 1e-6
    local = local - (num / den) * periodic

    fused = gate * periodic + (1.0 - gate) * 0.3 * local
    out_ref[0] = A + fused


@jax.jit
def kernel(x, W_dw, W_pw, gn_gamma, gn_beta, W_gate, res_gamma, W_loc_dw,
           W_loc_pw):
    B = x.shape[0]
    W_dw9 = W_dw[:, 0, :].T
    W_loc7 = W_loc_dw[:, 0, :].T
    res_row = res_gamma[:, :, 0, 0]
    gng = gn_gamma[None, :]
    gnb = gn_beta[None, :]

    full = lambda shape: pl.BlockSpec(shape, lambda b: (0,) * len(shape))
    return pl.pallas_call(
        _body,
        grid=(B,),
        in_specs=[
            pl.BlockSpec((1, T, C), lambda b: (b, 0, 0)),
            full((9, C)), full((C, C)), full((1, C)), full((1, C)),
            full((C, C)), full((1, C)), full((7, C)), full((C, C)),
        ],
        out_specs=pl.BlockSpec((1, T, C), lambda b: (b, 0, 0)),
        out_shape=jax.ShapeDtypeStruct((B, T, C), jnp.float32),
        compiler_params=pltpu.CompilerParams(
            dimension_semantics=("parallel",)),
    )(x, W_dw9, W_pw.T, gng, gnb, W_gate.T, res_row, W_loc7, W_loc_pw.T)


# speedup vs baseline: 1.3800x; 1.3800x over previous
import jax
import jax.numpy as jnp
from jax.experimental import pallas as pl
from jax.experimental.pallas import tpu as pltpu

T = 2048
C = 128
K = 3
CYC = 64
NB = 2
EPS_GN = 1e-5


def _erf(x):
    s = jnp.sign(x)
    a = jnp.abs(x)
    t = 1.0 / (1.0 + 0.3275911 * a)
    poly = ((((1.061405429 * t - 1.453152027) * t + 1.421413741) * t
             - 0.284496736) * t + 0.254829592) * t
    return s * (1.0 - poly * jnp.exp(-a * a))


def _gelu_exact(x):
    return 0.5 * x * (1.0 + _erf(x * 0.7071067811865476))


def _body(x_ref, wdw_ref, wpwt_ref, gng_ref, gnb_ref, wgt_ref, res_ref,
          wloc_ref, wlocpwt_ref, out_ref):
    W_dw9 = wdw_ref[...]
    W_pwT = wpwt_ref[...]
    gn_gamma = gng_ref[...]
    gn_beta = gnb_ref[...]
    W_gateT = wgt_ref[...]
    res_row = res_ref[...]
    W_loc7 = wloc_ref[...]
    W_loc_pwT = wlocpwt_ref[...]

    f32 = jnp.float32
    A = [x_ref[n] for n in range(NB)]

    S = []
    for n in range(NB):
        Apad = jnp.concatenate([A[n], A[n][:128]], axis=0)
        Sa = jnp.zeros((128, 256), f32)
        Sb = jnp.zeros((128, 256), f32)
        for i in range(0, 16, 2):
            Sa = Sa + jax.lax.dot_general(
                A[n][128 * i:128 * (i + 1)], Apad[128 * i:128 * i + 256],
                (((1,), (1,)), ((), ())), preferred_element_type=f32)
            Sb = Sb + jax.lax.dot_general(
                A[n][128 * (i + 1):128 * (i + 2)],
                Apad[128 * (i + 1):128 * (i + 1) + 256],
                (((1,), (1,)), ((), ())), preferred_element_type=f32)
        S.append(Sa + Sb)

    local_raw = []
    for n in range(NB):
        xpad = jnp.concatenate(
            [jnp.zeros((3, C), f32), A[n], jnp.zeros((3, C), f32)], axis=0)
        loc = jnp.zeros((T, C), f32)
        for j in range(7):
            loc = loc + xpad[j:j + T] * W_loc7[j][None, :]
        lp = jnp.dot(loc, W_loc_pwT, preferred_element_type=f32)
        local_raw.append(_gelu_exact(lp))

    lag = jax.lax.broadcasted_iota(jnp.int32, (1, 256), 1)
    row = jax.lax.broadcasted_iota(jnp.int32, (128, 256), 0)
    NEG = f32(-3e38)
    ids_n, ws_n, gate_n = [], [], []
    for n in range(NB):
        Sn = S[n]
        for k in range(7):
            s = 1 << k
            rolled = jnp.concatenate([Sn[:, s:], Sn[:, :s]], axis=1)
            Sn = jnp.where((row & s) != 0, rolled, Sn)
        r_full = jnp.sum(Sn, axis=0, keepdims=True) * (1.0 / C)
        rm = jnp.where((lag >= 32) & (lag <= 128), r_full, NEG)
        vs, ids = [], []
        for _ in range(K):
            v = jnp.max(rm, axis=1, keepdims=True)
            i_k = jnp.min(jnp.where(rm >= v, lag, 9999), axis=1, keepdims=True)
            rm = jnp.where(lag == i_k, NEG, rm)
            vs.append(v)
            ids.append(i_k)
        vmax = jnp.maximum(jnp.maximum(vs[0], vs[1]), vs[2])
        es = [jnp.exp(v - vmax) for v in vs]
        ssum = es[0] + es[1] + es[2]
        ws = [e / ssum for e in es]
        H = -(ws[0] * jnp.log(ws[0] + 1e-8) + ws[1] * jnp.log(ws[1] + 1e-8)
              + ws[2] * jnp.log(ws[2] + 1e-8))
        Hmax = jnp.log(f32(K) + 1e-8)
        Gamma = jnp.clip(1.0 - H / (Hmax + 1e-8), 0.0, 1.0)
        ids_n.append(ids)
        ws_n.append(ws)
        gate_n.append(jax.nn.sigmoid(4.0 * (Gamma - 0.5)))

    def _psel(r_i, ids):
        return jnp.where(r_i < CYC, ids[0],
                         jnp.where(r_i < 2 * CYC, ids[1], ids[2]))

    U_n = []
    for n in range(NB):
        ids = ids_n[n]
        r_i = jax.lax.broadcasted_iota(jnp.int32, (K * CYC, T), 0)
        t_i = jax.lax.broadcasted_iota(jnp.int32, (K * CYC, T), 1)
        p_b = _psel(r_i, ids)
        q = (r_i & (CYC - 1)) * p_b
        M = ((t_i >= q) & (t_i < q + p_b)).astype(f32)
        rr = jax.lax.broadcasted_iota(jnp.int32, (K * CYC, 128), 0)
        ss = jax.lax.broadcasted_iota(jnp.int32, (K * CYC, 128), 1)
        p_c = _psel(rr, ids)
        tpad_c = (jnp.ceil(2048.0 / p_c.astype(f32)) * p_c.astype(f32)
                  ).astype(jnp.int32)
        q_c = (rr & (CYC - 1)) * p_c
        tt = (2 * (T - 1) - (T - 128)) - ss
        refl = ((tt >= T) & (tt < tpad_c) & (tt >= q_c)
                & (tt < q_c + p_c)).astype(f32)
        U = (jnp.dot(M, A[n], preferred_element_type=f32)
             + jnp.dot(refl, A[n][T - 128:], preferred_element_type=f32))
        U_n.append(U * (1.0 / 128.0))

    g_of_c = jax.lax.broadcasted_iota(jnp.int32, (C, 32), 0) // 4
    g_id = jax.lax.broadcasted_iota(jnp.int32, (C, 32), 1)
    G = (g_of_c == g_id).astype(f32)
    X_n = []
    for n in range(NB):
        xks = []
        for k in range(K):
            u = U_n[n][CYC * k:CYC * (k + 1)]
            upad = jnp.concatenate(
                [jnp.zeros((4, C), f32), u, jnp.zeros((4, C), f32)], axis=0)
            xc = jnp.zeros((CYC, C), f32)
            for j in range(9):
                xc = xc + upad[j:j + CYC] * W_dw9[j][None, :]
            xp = jnp.dot(xc, W_pwT, preferred_element_type=f32)
            colsum = jnp.sum(xp, axis=0, keepdims=True)
            mu_g = jnp.dot(colsum, G) * (1.0 / 256.0)
            mu_row = jnp.dot(mu_g, G.T)
            dev = xp - mu_row
            var_g = jnp.dot(jnp.sum(dev * dev, axis=0, keepdims=True),
                            G) * (1.0 / 256.0)
            var_row = jnp.dot(var_g, G.T)
            xn = dev * jax.lax.rsqrt(var_row + EPS_GN)
            xn = xn * gn_gamma + gn_beta
            xg = _gelu_exact(xn)
            ubar = jnp.sum(u, axis=0, keepdims=True) * (1.0 / 64.0)
            gv = jax.nn.sigmoid(jnp.dot(ubar, W_gateT))
            xks.append(xg * (gv * (ws_n[n][k] * res_row)))
        X_n.append(jnp.concatenate(xks, axis=0))

    for n in range(NB):
        c_i = jax.lax.broadcasted_iota(jnp.int32, (T, K * CYC), 1)
        t2 = jax.lax.broadcasted_iota(jnp.int32, (T, K * CYC), 0)
        p_b = _psel(c_i, ids_n[n])
        q2 = (c_i & (CYC - 1)) * p_b
        MT = ((t2 >= q2) & (t2 < q2 + p_b)).astype(f32)
        periodic = A[n] + jnp.dot(MT, X_n[n], preferred_element_type=f32)
        local = local_raw[n]
        num = jnp.sum(local * periodic, axis=0, keepdims=True)
        den = jnp.sum(periodic * periodic, axis=0, keepdims=True) + 1e-6
        local = local - (num / den) * periodic
        gate = gate_n[n]
        fused = gate * periodic + (1.0 - gate) * 0.3 * local
        out_ref[n] = A[n] + fused


@jax.jit
def kernel(x, W_dw, W_pw, gn_gamma, gn_beta, W_gate, res_gamma, W_loc_dw,
           W_loc_pw):
    B = x.shape[0]
    W_dw9 = W_dw[:, 0, :].T
    W_loc7 = W_loc_dw[:, 0, :].T
    res_row = res_gamma[:, :, 0, 0]
    gng = gn_gamma[None, :]
    gnb = gn_beta[None, :]

    full = lambda shape: pl.BlockSpec(shape, lambda b: (0,) * len(shape))
    return pl.pallas_call(
        _body,
        grid=(B // NB,),
        in_specs=[
            pl.BlockSpec((NB, T, C), lambda b: (b, 0, 0)),
            full((9, C)), full((C, C)), full((1, C)), full((1, C)),
            full((C, C)), full((1, C)), full((7, C)), full((C, C)),
        ],
        out_specs=pl.BlockSpec((NB, T, C), lambda b: (b, 0, 0)),
        out_shape=jax.ShapeDtypeStruct((B, T, C), jnp.float32),
        compiler_params=pltpu.CompilerParams(
            dimension_semantics=("parallel",)),
    )(x, W_dw9, W_pw.T, gng, gnb, W_gate.T, res_row, W_loc7, W_loc_pw.T)


# speedup vs baseline: 1.5368x; 1.1136x over previous
import jax
import jax.numpy as jnp
from jax.experimental import pallas as pl
from jax.experimental.pallas import tpu as pltpu

T = 2048
C = 128
K = 3
CYC = 64
NB = 2
EPS_GN = 1e-5


def _erf(x):
    s = jnp.sign(x)
    a = jnp.abs(x)
    t = 1.0 / (1.0 + 0.3275911 * a)
    poly = ((((1.061405429 * t - 1.453152027) * t + 1.421413741) * t
             - 0.284496736) * t + 0.254829592) * t
    return s * (1.0 - poly * jnp.exp(-a * a))


def _gelu_exact(x):
    return 0.5 * x * (1.0 + _erf(x * 0.7071067811865476))


def _body(x_ref, wdw_ref, wpwt_ref, gng_ref, gnb_ref, wgt_ref, res_ref,
          wloc_ref, out_ref):
    W_dw9 = wdw_ref[...]
    W_pwT = wpwt_ref[...]
    gn_gamma = gng_ref[...]
    gn_beta = gnb_ref[...]
    W_gateT = wgt_ref[...]
    res_row = res_ref[...]
    W_loc7 = wloc_ref[...]

    f32 = jnp.float32
    A = [x_ref[n] for n in range(NB)]

    S = []
    for n in range(NB):
        Apad = jnp.concatenate([A[n], A[n][:128]], axis=0)
        Sa = jnp.zeros((128, 256), f32)
        Sb = jnp.zeros((128, 256), f32)
        for i in range(0, 16, 2):
            Sa = Sa + jax.lax.dot_general(
                A[n][128 * i:128 * (i + 1)], Apad[128 * i:128 * i + 256],
                (((1,), (1,)), ((), ())), preferred_element_type=f32)
            Sb = Sb + jax.lax.dot_general(
                A[n][128 * (i + 1):128 * (i + 2)],
                Apad[128 * (i + 1):128 * (i + 1) + 256],
                (((1,), (1,)), ((), ())), preferred_element_type=f32)
        S.append(Sa + Sb)

    local_raw = []
    for n in range(NB):
        xpadb = jnp.concatenate(
            [jnp.zeros((3, C), jnp.bfloat16), A[n].astype(jnp.bfloat16),
             jnp.zeros((3, C), jnp.bfloat16)], axis=0)
        lp = jnp.zeros((T, C), f32)
        for j in range(7):
            lp = lp + jnp.dot(xpadb[j:j + T], W_loc7[C * j:C * (j + 1)],
                              preferred_element_type=f32)
        local_raw.append(_gelu_exact(lp))

    lag = jax.lax.broadcasted_iota(jnp.int32, (1, 256), 1)
    row = jax.lax.broadcasted_iota(jnp.int32, (128, 256), 0)
    NEG = f32(-3e38)
    ids_n, ws_n, gate_n = [], [], []
    for n in range(NB):
        Sn = S[n]
        for k in range(7):
            s = 1 << k
            rolled = jnp.concatenate([Sn[:, s:], Sn[:, :s]], axis=1)
            Sn = jnp.where((row & s) != 0, rolled, Sn)
        r_full = jnp.sum(Sn, axis=0, keepdims=True) * (1.0 / C)
        rm = jnp.where((lag >= 32) & (lag <= 128), r_full, NEG)
        vs, ids = [], []
        for _ in range(K):
            v = jnp.max(rm, axis=1, keepdims=True)
            i_k = jnp.min(jnp.where(rm >= v, lag, 9999), axis=1, keepdims=True)
            rm = jnp.where(lag == i_k, NEG, rm)
            vs.append(v)
            ids.append(i_k)
        vmax = jnp.maximum(jnp.maximum(vs[0], vs[1]), vs[2])
        es = [jnp.exp(v - vmax) for v in vs]
        ssum = es[0] + es[1] + es[2]
        ws = [e / ssum for e in es]
        H = -(ws[0] * jnp.log(ws[0] + 1e-8) + ws[1] * jnp.log(ws[1] + 1e-8)
              + ws[2] * jnp.log(ws[2] + 1e-8))
        Hmax = jnp.log(f32(K) + 1e-8)
        Gamma = jnp.clip(1.0 - H / (Hmax + 1e-8), 0.0, 1.0)
        ids_n.append(ids)
        ws_n.append(ws)
        gate_n.append(jax.nn.sigmoid(4.0 * (Gamma - 0.5)))

    def _psel(r_i, ids):
        return jnp.where(r_i < CYC, ids[0],
                         jnp.where(r_i < 2 * CYC, ids[1], ids[2]))

    U_n, M_n = [], []
    for n in range(NB):
        ids = ids_n[n]
        r_c = jax.lax.broadcasted_iota(jnp.int32, (K * CYC, 1), 0)
        t_i = jax.lax.broadcasted_iota(jnp.int32, (K * CYC, T), 1)
        p_c1 = _psel(r_c, ids)
        q_c1 = (r_c & (CYC - 1)) * p_c1
        M = ((t_i >= q_c1) & (t_i < q_c1 + p_c1)).astype(f32)
        M_n.append(M)
        rr = jax.lax.broadcasted_iota(jnp.int32, (K * CYC, 128), 0)
        ss = jax.lax.broadcasted_iota(jnp.int32, (K * CYC, 128), 1)
        p_c = _psel(rr, ids)
        tpad_c = (jnp.ceil(2048.0 / p_c.astype(f32)) * p_c.astype(f32)
                  ).astype(jnp.int32)
        q_c = (rr & (CYC - 1)) * p_c
        tt = (2 * (T - 1) - (T - 128)) - ss
        refl = ((tt >= T) & (tt < tpad_c) & (tt >= q_c)
                & (tt < q_c + p_c)).astype(f32)
        U = (jnp.dot(M, A[n], preferred_element_type=f32)
             + jnp.dot(refl, A[n][T - 128:], preferred_element_type=f32))
        U_n.append(U * (1.0 / 128.0))

    g_of_c = jax.lax.broadcasted_iota(jnp.int32, (C, 32), 0) // 4
    g_id = jax.lax.broadcasted_iota(jnp.int32, (C, 32), 1)
    G = (g_of_c == g_id).astype(f32)
    X_n = []
    for n in range(NB):
        xks = []
        for k in range(K):
            u = U_n[n][CYC * k:CYC * (k + 1)]
            upad = jnp.concatenate(
                [jnp.zeros((4, C), f32), u, jnp.zeros((4, C), f32)], axis=0)
            xc = jnp.zeros((CYC, C), f32)
            for j in range(9):
                xc = xc + upad[j:j + CYC] * W_dw9[j][None, :]
            xp = jnp.dot(xc, W_pwT, preferred_element_type=f32)
            colsum = jnp.sum(xp, axis=0, keepdims=True)
            mu_g = jnp.dot(colsum, G) * (1.0 / 256.0)
            mu_row = jnp.dot(mu_g, G.T)
            dev = xp - mu_row
            var_g = jnp.dot(jnp.sum(dev * dev, axis=0, keepdims=True),
                            G) * (1.0 / 256.0)
            var_row = jnp.dot(var_g, G.T)
            xn = dev * jax.lax.rsqrt(var_row + EPS_GN)
            xn = xn * gn_gamma + gn_beta
            xg = _gelu_exact(xn)
            ubar = jnp.sum(u, axis=0, keepdims=True) * (1.0 / 64.0)
            gv = jax.nn.sigmoid(jnp.dot(ubar, W_gateT))
            xks.append(xg * (gv * (ws_n[n][k] * res_row)))
        X_n.append(jnp.concatenate(xks, axis=0))

    for n in range(NB):
        periodic = A[n] + jax.lax.dot_general(
            M_n[n], X_n[n], (((0,), (0,)), ((), ())),
            preferred_element_type=f32)
        local = local_raw[n]
        num = jnp.sum(local * periodic, axis=0, keepdims=True)
        den = jnp.sum(periodic * periodic, axis=0, keepdims=True) + 1e-6
        local = local - (num / den) * periodic
        gate = gate_n[n]
        fused = gate * periodic + (1.0 - gate) * 0.3 * local
        out_ref[n] = A[n] + fused


@jax.jit
def kernel(x, W_dw, W_pw, gn_gamma, gn_beta, W_gate, res_gamma, W_loc_dw,
           W_loc_pw):
    B = x.shape[0]
    W_dw9 = W_dw[:, 0, :].T
    W_locj = (W_loc_dw[:, 0, :].T[:, :, None]
              * W_loc_pw.T[None, :, :]).reshape(7 * C, C).astype(jnp.bfloat16)
    res_row = res_gamma[:, :, 0, 0]
    gng = gn_gamma[None, :]
    gnb = gn_beta[None, :]

    full = lambda shape: pl.BlockSpec(shape, lambda b: (0,) * len(shape))
    return pl.pallas_call(
        _body,
        grid=(B // NB,),
        in_specs=[
            pl.BlockSpec((NB, T, C), lambda b: (b, 0, 0)),
            full((9, C)), full((C, C)), full((1, C)), full((1, C)),
            full((C, C)), full((1, C)), full((7 * C, C)),
        ],
        out_specs=pl.BlockSpec((NB, T, C), lambda b: (b, 0, 0)),
        out_shape=jax.ShapeDtypeStruct((B, T, C), jnp.float32),
        compiler_params=pltpu.CompilerParams(
            dimension_semantics=("parallel",),
            fuse_transposed_lhs_in_matmul=True),
    )(x, W_dw9, W_pw.T, gng, gnb, W_gate.T, res_row, W_locj)


# speedup vs baseline: 1.7005x; 1.1066x over previous
import jax
import jax.numpy as jnp
from jax.experimental import pallas as pl
from jax.experimental.pallas import tpu as pltpu

T = 2048
C = 128
K = 3
CYC = 64
NB = 2
EPS_GN = 1e-5


def _gelu(x):
    return 0.5 * x * (1.0 + jnp.tanh(0.7978845608028654 * (x + 0.044715 * x * x * x)))


def _body(x_ref, wdw_ref, wpwt_ref, gng_ref, gnb_ref, wgt_ref, res_ref,
          wloc_ref, out_ref):
    W_dw9 = wdw_ref[...]
    W_pwT = wpwt_ref[...]
    gn_gamma = gng_ref[...]
    gn_beta = gnb_ref[...]
    W_gateT = wgt_ref[...]
    res_row = res_ref[...]
    W_loc7 = wloc_ref[...]

    f32 = jnp.float32
    A = [x_ref[n] for n in range(NB)]

    S = []
    for n in range(NB):
        Apad = jnp.concatenate([A[n], A[n][:128]], axis=0)
        Sa = jnp.zeros((128, 256), f32)
        Sb = jnp.zeros((128, 256), f32)
        for i in range(0, 16, 2):
            Sa = Sa + jax.lax.dot_general(
                A[n][128 * i:128 * (i + 1)], Apad[128 * i:128 * i + 256],
                (((1,), (1,)), ((), ())), preferred_element_type=f32)
            Sb = Sb + jax.lax.dot_general(
                A[n][128 * (i + 1):128 * (i + 2)],
                Apad[128 * (i + 1):128 * (i + 1) + 256],
                (((1,), (1,)), ((), ())), preferred_element_type=f32)
        S.append(Sa + Sb)

    local_raw = []
    for n in range(NB):
        xpadb = jnp.concatenate(
            [jnp.zeros((3, C), jnp.bfloat16), A[n].astype(jnp.bfloat16),
             jnp.zeros((3, C), jnp.bfloat16)], axis=0)
        lp = jnp.zeros((T, C), f32)
        for j in range(7):
            lp = lp + jnp.dot(xpadb[j:j + T], W_loc7[C * j:C * (j + 1)],
                              preferred_element_type=f32)
        local_raw.append(_gelu(lp))

    lag = jax.lax.broadcasted_iota(jnp.int32, (1, 256), 1)
    row = jax.lax.broadcasted_iota(jnp.int32, (128, 256), 0)
    NEG = f32(-3e38)
    ids_n, ws_n, gate_n = [], [], []
    for n in range(NB):
        Sn = S[n]
        for k in range(7):
            s = 1 << k
            rolled = jnp.concatenate([Sn[:, s:], Sn[:, :s]], axis=1)
            Sn = jnp.where((row & s) != 0, rolled, Sn)
        r_full = jnp.sum(Sn, axis=0, keepdims=True) * (1.0 / C)
        rm = jnp.where((lag >= 32) & (lag <= 128), r_full, NEG)
        vs, ids = [], []
        for _ in range(K):
            v = jnp.max(rm, axis=1, keepdims=True)
            i_k = jnp.min(jnp.where(rm >= v, lag, 9999), axis=1, keepdims=True)
            rm = jnp.where(lag == i_k, NEG, rm)
            vs.append(v)
            ids.append(i_k)
        vmax = jnp.maximum(jnp.maximum(vs[0], vs[1]), vs[2])
        es = [jnp.exp(v - vmax) for v in vs]
        ssum = es[0] + es[1] + es[2]
        ws = [e / ssum for e in es]
        H = -(ws[0] * jnp.log(ws[0] + 1e-8) + ws[1] * jnp.log(ws[1] + 1e-8)
              + ws[2] * jnp.log(ws[2] + 1e-8))
        Hmax = jnp.log(f32(K) + 1e-8)
        Gamma = jnp.clip(1.0 - H / (Hmax + 1e-8), 0.0, 1.0)
        ids_n.append(ids)
        ws_n.append(ws)
        gate_n.append(jax.nn.sigmoid(4.0 * (Gamma - 0.5)))

    def _psel(r_i, ids):
        return jnp.where(r_i < CYC, ids[0],
                         jnp.where(r_i < 2 * CYC, ids[1], ids[2]))

    U_n, M_n = [], []
    for n in range(NB):
        ids = ids_n[n]
        r_c = jax.lax.broadcasted_iota(jnp.int32, (K * CYC, 1), 0)
        t_i = jax.lax.broadcasted_iota(jnp.int32, (K * CYC, T), 1)
        p_c1 = _psel(r_c, ids)
        q_c1 = (r_c & (CYC - 1)) * p_c1
        M = ((t_i >= q_c1) & (t_i < q_c1 + p_c1)).astype(f32)
        M_n.append(M)
        rr = jax.lax.broadcasted_iota(jnp.int32, (K * CYC, 128), 0)
        ss = jax.lax.broadcasted_iota(jnp.int32, (K * CYC, 128), 1)
        p_c = _psel(rr, ids)
        tpad_c = (jnp.ceil(2048.0 / p_c.astype(f32)) * p_c.astype(f32)
                  ).astype(jnp.int32)
        q_c = (rr & (CYC - 1)) * p_c
        tt = (2 * (T - 1) - (T - 128)) - ss
        refl = ((tt >= T) & (tt < tpad_c) & (tt >= q_c)
                & (tt < q_c + p_c)).astype(f32)
        U = (jnp.dot(M, A[n], preferred_element_type=f32)
             + jnp.dot(refl, A[n][T - 128:], preferred_element_type=f32))
        U_n.append(U * (1.0 / 128.0))

    g_of_c = jax.lax.broadcasted_iota(jnp.int32, (C, 32), 0) // 4
    g_id = jax.lax.broadcasted_iota(jnp.int32, (C, 32), 1)
    G = (g_of_c == g_id).astype(f32)
    X_n = []
    for n in range(NB):
        U = U_n[n]
        z4 = jnp.zeros((4, C), f32)
        z16 = jnp.zeros((16, C), f32)
        Up = jnp.concatenate(
            [z4, U[0:CYC], z16, U[CYC:2 * CYC], z16, U[2 * CYC:3 * CYC], z4],
            axis=0)
        xc = jnp.zeros((224, C), f32)
        for j in range(9):
            xc = xc + Up[j:j + 224] * W_dw9[j][None, :]
        xp = jnp.dot(xc, W_pwT, preferred_element_type=f32)
        xps = [xp[80 * k:80 * k + CYC] for k in range(K)]
        CS = jnp.concatenate(
            [jnp.sum(s, axis=0, keepdims=True) for s in xps], axis=0)
        MU = jnp.dot(jnp.dot(CS, G), G.T) * (1.0 / 256.0)
        devs = [xps[k] - MU[k:k + 1] for k in range(K)]
        VS = jnp.concatenate(
            [jnp.sum(d * d, axis=0, keepdims=True) for d in devs], axis=0)
        VR = jnp.dot(jnp.dot(VS, G), G.T) * (1.0 / 256.0)
        UB = jnp.concatenate(
            [jnp.sum(U[CYC * k:CYC * (k + 1)], axis=0, keepdims=True)
             for k in range(K)], axis=0) * (1.0 / 64.0)
        GV = jax.nn.sigmoid(jnp.dot(UB, W_gateT))
        xks = []
        for k in range(K):
            xn = devs[k] * jax.lax.rsqrt(VR[k:k + 1] + EPS_GN)
            xn = xn * gn_gamma + gn_beta
            xg = _gelu(xn)
            xks.append(xg * (GV[k:k + 1] * (ws_n[n][k] * res_row)))
        X_n.append(jnp.concatenate(xks, axis=0))

    for n in range(NB):
        periodic = A[n] + jax.lax.dot_general(
            M_n[n], X_n[n], (((0,), (0,)), ((), ())),
            preferred_element_type=f32)
        local = local_raw[n]
        num = jnp.sum(local * periodic, axis=0, keepdims=True)
        den = jnp.sum(periodic * periodic, axis=0, keepdims=True) + 1e-6
        local = local - (num / den) * periodic
        gate = gate_n[n]
        fused = gate * periodic + (1.0 - gate) * 0.3 * local
        out_ref[n] = A[n] + fused


@jax.jit
def kernel(x, W_dw, W_pw, gn_gamma, gn_beta, W_gate, res_gamma, W_loc_dw,
           W_loc_pw):
    B = x.shape[0]
    W_dw9 = W_dw[:, 0, :].T
    W_locj = (W_loc_dw[:, 0, :].T[:, :, None]
              * W_loc_pw.T[None, :, :]).reshape(7 * C, C).astype(jnp.bfloat16)
    res_row = res_gamma[:, :, 0, 0]
    gng = gn_gamma[None, :]
    gnb = gn_beta[None, :]

    full = lambda shape: pl.BlockSpec(shape, lambda b: (0,) * len(shape))
    return pl.pallas_call(
        _body,
        grid=(B // NB,),
        in_specs=[
            pl.BlockSpec((NB, T, C), lambda b: (b, 0, 0)),
            full((9, C)), full((C, C)), full((1, C)), full((1, C)),
            full((C, C)), full((1, C)), full((7 * C, C)),
        ],
        out_specs=pl.BlockSpec((NB, T, C), lambda b: (b, 0, 0)),
        out_shape=jax.ShapeDtypeStruct((B, T, C), jnp.float32),
        compiler_params=pltpu.CompilerParams(
            dimension_semantics=("parallel",),
            fuse_transposed_lhs_in_matmul=True),
    )(x, W_dw9, W_pw.T, gng, gnb, W_gate.T, res_row, W_locj)


# speedup vs baseline: 1.7056x; 1.0030x over previous
import jax
import jax.numpy as jnp
from jax.experimental import pallas as pl
from jax.experimental.pallas import tpu as pltpu

T = 2048
C = 128
K = 3
CYC = 64
NB = 4
EPS_GN = 1e-5


def _gelu(x):
    return 0.5 * x * (1.0 + jnp.tanh(0.7978845608028654 * (x + 0.044715 * x * x * x)))


def _body(x_ref, wdw_ref, wpwt_ref, gng_ref, gnb_ref, wgt_ref, res_ref,
          wloc_ref, out_ref):
    W_dw9 = wdw_ref[...]
    W_pwT = wpwt_ref[...]
    gn_gamma = gng_ref[...]
    gn_beta = gnb_ref[...]
    W_gateT = wgt_ref[...]
    res_row = res_ref[...]
    W_loc7 = wloc_ref[...]

    f32 = jnp.float32
    A = [x_ref[n] for n in range(NB)]

    S = []
    for n in range(NB):
        Apad = jnp.concatenate([A[n], A[n][:128]], axis=0)
        Sa = jnp.zeros((128, 256), f32)
        Sb = jnp.zeros((128, 256), f32)
        for i in range(0, 16, 2):
            Sa = Sa + jax.lax.dot_general(
                A[n][128 * i:128 * (i + 1)], Apad[128 * i:128 * i + 256],
                (((1,), (1,)), ((), ())), preferred_element_type=f32)
            Sb = Sb + jax.lax.dot_general(
                A[n][128 * (i + 1):128 * (i + 2)],
                Apad[128 * (i + 1):128 * (i + 1) + 256],
                (((1,), (1,)), ((), ())), preferred_element_type=f32)
        S.append(Sa + Sb)

    local_raw = []
    for n in range(NB):
        xpadb = jnp.concatenate(
            [jnp.zeros((3, C), jnp.bfloat16), A[n].astype(jnp.bfloat16),
             jnp.zeros((3, C), jnp.bfloat16)], axis=0)
        lp = jnp.zeros((T, C), f32)
        for j in range(7):
            lp = lp + jnp.dot(xpadb[j:j + T], W_loc7[C * j:C * (j + 1)],
                              preferred_element_type=f32)
        local_raw.append(_gelu(lp))

    lag = jax.lax.broadcasted_iota(jnp.int32, (1, 256), 1)
    row = jax.lax.broadcasted_iota(jnp.int32, (128, 256), 0)
    NEG = f32(-3e38)
    ids_n, ws_n, gate_n = [], [], []
    for n in range(NB):
        Sn = S[n]
        for k in range(7):
            s = 1 << k
            rolled = jnp.concatenate([Sn[:, s:], Sn[:, :s]], axis=1)
            Sn = jnp.where((row & s) != 0, rolled, Sn)
        r_full = jnp.sum(Sn, axis=0, keepdims=True) * (1.0 / C)
        rm = jnp.where((lag >= 32) & (lag <= 128), r_full, NEG)
        vs, ids = [], []
        for _ in range(K):
            v = jnp.max(rm, axis=1, keepdims=True)
            i_k = jnp.min(jnp.where(rm >= v, lag, 9999), axis=1, keepdims=True)
            rm = jnp.where(lag == i_k, NEG, rm)
            vs.append(v)
            ids.append(i_k)
        vmax = jnp.maximum(jnp.maximum(vs[0], vs[1]), vs[2])
        es = [jnp.exp(v - vmax) for v in vs]
        ssum = es[0] + es[1] + es[2]
        ws = [e / ssum for e in es]
        H = -(ws[0] * jnp.log(ws[0] + 1e-8) + ws[1] * jnp.log(ws[1] + 1e-8)
              + ws[2] * jnp.log(ws[2] + 1e-8))
        Hmax = jnp.log(f32(K) + 1e-8)
        Gamma = jnp.clip(1.0 - H / (Hmax + 1e-8), 0.0, 1.0)
        ids_n.append(ids)
        ws_n.append(ws)
        gate_n.append(jax.nn.sigmoid(4.0 * (Gamma - 0.5)))

    def _psel(r_i, ids):
        return jnp.where(r_i < CYC, ids[0],
                         jnp.where(r_i < 2 * CYC, ids[1], ids[2]))

    U_n, M_n = [], []
    for n in range(NB):
        ids = ids_n[n]
        r_c = jax.lax.broadcasted_iota(jnp.int32, (K * CYC, 1), 0)
        t_i = jax.lax.broadcasted_iota(jnp.int32, (K * CYC, T), 1)
        p_c1 = _psel(r_c, ids)
        q_c1 = (r_c & (CYC - 1)) * p_c1
        M = ((t_i >= q_c1) & (t_i < q_c1 + p_c1)).astype(f32)
        M_n.append(M)
        rr = jax.lax.broadcasted_iota(jnp.int32, (K * CYC, 128), 0)
        ss = jax.lax.broadcasted_iota(jnp.int32, (K * CYC, 128), 1)
        p_c = _psel(rr, ids)
        tpad_c = (jnp.ceil(2048.0 / p_c.astype(f32)) * p_c.astype(f32)
                  ).astype(jnp.int32)
        q_c = (rr & (CYC - 1)) * p_c
        tt = (2 * (T - 1) - (T - 128)) - ss
        refl = ((tt >= T) & (tt < tpad_c) & (tt >= q_c)
                & (tt < q_c + p_c)).astype(f32)
        U = (jnp.dot(M, A[n], preferred_element_type=f32)
             + jnp.dot(refl, A[n][T - 128:], preferred_element_type=f32))
        U_n.append(U * (1.0 / 128.0))

    g_of_c = jax.lax.broadcasted_iota(jnp.int32, (C, 32), 0) // 4
    g_id = jax.lax.broadcasted_iota(jnp.int32, (C, 32), 1)
    G = (g_of_c == g_id).astype(f32)
    X_n = []
    for n in range(NB):
        U = U_n[n]
        z4 = jnp.zeros((4, C), f32)
        z16 = jnp.zeros((16, C), f32)
        Up = jnp.concatenate(
            [z4, U[0:CYC], z16, U[CYC:2 * CYC], z16, U[2 * CYC:3 * CYC], z4],
            axis=0)
        xc = jnp.zeros((224, C), f32)
        for j in range(9):
            xc = xc + Up[j:j + 224] * W_dw9[j][None, :]
        xp = jnp.dot(xc, W_pwT, preferred_element_type=f32)
        xps = [xp[80 * k:80 * k + CYC] for k in range(K)]
        CS = jnp.concatenate(
            [jnp.sum(s, axis=0, keepdims=True) for s in xps], axis=0)
        MU = jnp.dot(jnp.dot(CS, G), G.T) * (1.0 / 256.0)
        devs = [xps[k] - MU[k:k + 1] for k in range(K)]
        VS = jnp.concatenate(
            [jnp.sum(d * d, axis=0, keepdims=True) for d in devs], axis=0)
        VR = jnp.dot(jnp.dot(VS, G), G.T) * (1.0 / 256.0)
        UB = jnp.concatenate(
            [jnp.sum(U[CYC * k:CYC * (k + 1)], axis=0, keepdims=True)
             for k in range(K)], axis=0) * (1.0 / 64.0)
        GV = jax.nn.sigmoid(jnp.dot(UB, W_gateT))
        xks = []
        for k in range(K):
            xn = devs[k] * jax.lax.rsqrt(VR[k:k + 1] + EPS_GN)
            xn = xn * gn_gamma + gn_beta
            xg = _gelu(xn)
            xks.append(xg * (GV[k:k + 1] * (ws_n[n][k] * res_row)))
        X_n.append(jnp.concatenate(xks, axis=0))

    for n in range(NB):
        periodic = A[n] + jax.lax.dot_general(
            M_n[n], X_n[n], (((0,), (0,)), ((), ())),
            preferred_element_type=f32)
        local = local_raw[n]
        num = jnp.sum(local * periodic, axis=0, keepdims=True)
        den = jnp.sum(periodic * periodic, axis=0, keepdims=True) + 1e-6
        local = local - (num / den) * periodic
        gate = gate_n[n]
        fused = gate * periodic + (1.0 - gate) * 0.3 * local
        out_ref[n] = A[n] + fused


@jax.jit
def kernel(x, W_dw, W_pw, gn_gamma, gn_beta, W_gate, res_gamma, W_loc_dw,
           W_loc_pw):
    B = x.shape[0]
    W_dw9 = W_dw[:, 0, :].T
    W_locj = (W_loc_dw[:, 0, :].T[:, :, None]
              * W_loc_pw.T[None, :, :]).reshape(7 * C, C).astype(jnp.bfloat16)
    res_row = res_gamma[:, :, 0, 0]
    gng = gn_gamma[None, :]
    gnb = gn_beta[None, :]

    full = lambda shape: pl.BlockSpec(shape, lambda b: (0,) * len(shape))
    return pl.pallas_call(
        _body,
        grid=(B // NB,),
        in_specs=[
            pl.BlockSpec((NB, T, C), lambda b: (b, 0, 0)),
            full((9, C)), full((C, C)), full((1, C)), full((1, C)),
            full((C, C)), full((1, C)), full((7 * C, C)),
        ],
        out_specs=pl.BlockSpec((NB, T, C), lambda b: (b, 0, 0)),
        out_shape=jax.ShapeDtypeStruct((B, T, C), jnp.float32),
        compiler_params=pltpu.CompilerParams(
            dimension_semantics=("parallel",),
            fuse_transposed_lhs_in_matmul=True),
    )(x, W_dw9, W_pw.T, gng, gnb, W_gate.T, res_row, W_locj)
